# baseline (speedup 1.0000x reference)
import functools

import jax
import jax.numpy as jnp
from jax import lax
from jax.experimental import pallas as pl
from jax.experimental.pallas import tpu as pltpu
from jax.experimental.pallas import tpu_sc as plsc

N = 10000
E = 320000
D = 128
BN_EPS = 1e-5

NC = 2
NS = 16
NW = NC * NS
ROWS_PER_W = 320
OWNER_MAGIC = 13108
DST_SENT = N
SCAN_SENT = 1 << 15
BLK = 4096
NB = -(-E // BLK)
SCAN_LEN = NB * BLK
FLUSH = 1600
SEL_CAP = FLUSH + BLK + 64
CH = 80
CAP = E

_sc_mesh = plsc.VectorSubcoreMesh(
    core_axis_name="c", subcore_axis_name="s", num_cores=NC, num_subcores=NS
)


@functools.partial(
    pl.kernel,
    out_type=(
        jax.ShapeDtypeStruct((NW * CAP,), jnp.int32),
        jax.ShapeDtypeStruct((NW, 16), jnp.int32),
    ),
    mesh=_sc_mesh,
    scratch_types=[
        pltpu.VMEM((2, BLK), jnp.int32),
        pltpu.VMEM((2, BLK), jnp.int32),
        pltpu.VMEM((SEL_CAP,), jnp.int32),
        pltpu.VMEM((16,), jnp.int32),
        pltpu.SemaphoreType.DMA,
        pltpu.SemaphoreType.DMA,
    ],
    compiler_params=pltpu.CompilerParams(needs_layout_passes=False),
)
def _sc_prepass(ssc, dsc, plist, counts, sbuf, dbuf, sel, cbuf, sem_s, sem_d):
    c = lax.axis_index("c")
    s = lax.axis_index("s")
    w = c * NS + s
    base = w * CAP

    def start_blk(b, slot):
        pltpu.async_copy(ssc.at[pl.ds(b * BLK, BLK)], sbuf.at[slot], sem_s)
        pltpu.async_copy(dsc.at[pl.ds(b * BLK, BLK)], dbuf.at[slot], sem_d)

    def wait_blk(slot):
        pltpu.make_async_copy(ssc.at[pl.ds(0, BLK)], sbuf.at[slot], sem_s).wait()
        pltpu.make_async_copy(dsc.at[pl.ds(0, BLK)], dbuf.at[slot], sem_d).wait()

    start_blk(0, 0)

    def block(b, carry):
        off, nfl = carry
        slot = lax.rem(b, 2)

        @pl.when(b + 1 < NB)
        def _prefetch():
            start_blk(b + 1, lax.rem(b + 1, 2))

        wait_blk(slot)

        def scan16(j, off):
            d16 = dbuf[slot, pl.ds(j * 16, 16)]
            s16 = sbuf[slot, pl.ds(j * 16, 16)]
            ow = lax.shift_right_logical(d16 * OWNER_MAGIC, 22)
            m = ow == w
            lane = lax.iota(jnp.int32, 16)
            pk = jnp.bitwise_or(lax.shift_left(s16, 14), d16)
            zero = jnp.zeros((16,), jnp.int32)
            mi = jnp.where(m, jnp.full((16,), 1, jnp.int32), zero)
            csum = plsc.cumsum(mi)
            pos = csum - mi + off
            plsc.store_scatter(sel, [pos], pk, mask=m)
            pc = plsc.all_reduce_population_count(m)
            return off + pc[0]

        off = lax.fori_loop(0, BLK // 16, scan16, off)

        def do_flush(carry):
            off, nfl = carry
            pltpu.sync_copy(sel.at[pl.ds(0, FLUSH)],
                            plist.at[pl.ds(base + nfl * FLUSH, FLUSH)])
            nmv = lax.shift_right_logical(off - FLUSH + 15, 4)

            def mv(i, _):
                sel[pl.ds(i * 16, 16)] = sel[pl.ds(FLUSH + i * 16, 16)]
                return 0

            lax.fori_loop(0, nmv, mv, 0)
            return off - FLUSH, nfl + 1

        off, nfl = lax.while_loop(lambda cr: cr[0] >= FLUSH, do_flush,
                                  (off, nfl))
        return off, nfl

    off, nfl = lax.fori_loop(0, NB, block, (jnp.int32(0), jnp.int32(0)))

    @pl.when(off > 0)
    def _tail():
        start = off & ~jnp.int32(15)
        lane = lax.iota(jnp.int32, 16)
        vp = jnp.where(lane >= off - start, jnp.int32(DST_SENT),
                       sel[pl.ds(start, 16)])
        sel[pl.ds(start, 16)] = vp

        def fill(i, _):
            sel[pl.ds(start + 16 + i * 16, 16)] = jnp.full(
                (16,), DST_SENT, jnp.int32)
            return 0

        lax.fori_loop(0, lax.shift_right_logical(FLUSH - start - 16, 4),
                      fill, 0)
        pltpu.sync_copy(sel.at[pl.ds(0, FLUSH)],
                        plist.at[pl.ds(base + nfl * FLUSH, FLUSH)])

    total = nfl + jnp.where(off > 0, jnp.int32(1), jnp.int32(0))
    cbuf[...] = jnp.full((16,), total, jnp.int32)
    pltpu.sync_copy(cbuf, counts.at[w])


@functools.partial(
    pl.kernel,
    out_type=jax.ShapeDtypeStruct((N, D), jnp.float32),
    mesh=_sc_mesh,
    scratch_types=[
        pltpu.VMEM((CH,), jnp.int32),
        pltpu.VMEM((CH,), jnp.int32),
        pltpu.VMEM((CH,), jnp.int32),
        pltpu.VMEM((CH, D), jnp.float32),
        pltpu.VMEM((16,), jnp.int32),
        pltpu.VMEM_SHARED((N + 16, D), jnp.float32),
        pltpu.SemaphoreType.DMA,
    ],
    compiler_params=pltpu.CompilerParams(needs_layout_passes=False),
)
def _sc_agg(x_hbm, plist, counts, out_hbm, pbuf, srcv, dstv, rows, cbuf,
            agg, sem):
    c = lax.axis_index("c")
    s = lax.axis_index("s")
    w = c * NS + s

    def zrow(i, _):
        def zcol(j, _):
            rows[i, pl.ds(j * 16, 16)] = jnp.zeros((16,), jnp.float32)
            return 0
        return lax.fori_loop(0, D // 16, zcol, 0)

    lax.fori_loop(0, CH, zrow, 0)
    r0 = w * ROWS_PER_W

    @pl.when(w < NW - 1)
    def _zero_full():
        def zc(k, _):
            pltpu.sync_copy(rows, agg.at[pl.ds(r0 + k * CH, CH)])
            return 0
        lax.fori_loop(0, ROWS_PER_W // CH, zc, 0)

    @pl.when(w == NW - 1)
    def _zero_last():
        pltpu.sync_copy(rows, agg.at[pl.ds(r0, CH)])

    plsc.subcore_barrier()

    pltpu.sync_copy(counts.at[w], cbuf)
    nchunk = cbuf[...][0] * (FLUSH // CH)
    base = w * CAP

    def body(k, _):
        o = base + k * CH
        pltpu.sync_copy(plist.at[pl.ds(o, CH)], pbuf)

        def unpack(j, _):
            v = pbuf[pl.ds(j * 16, 16)]
            srcv[pl.ds(j * 16, 16)] = lax.shift_right_logical(v, 14)
            dstv[pl.ds(j * 16, 16)] = jnp.bitwise_and(v, 16383)
            return 0

        lax.fori_loop(0, CH // 16, unpack, 0)
        pltpu.async_copy(x_hbm.at[srcv], rows, sem).wait()
        pltpu.sync_copy(rows, agg.at[dstv], add=True)
        return 0

    lax.fori_loop(0, nchunk, body, 0)

    plsc.subcore_barrier()

    @pl.when(w < NW - 1)
    def _out_full():
        pltpu.sync_copy(agg.at[pl.ds(r0, ROWS_PER_W)],
                        out_hbm.at[pl.ds(r0, ROWS_PER_W)])

    @pl.when(w == NW - 1)
    def _out_last():
        pltpu.sync_copy(agg.at[pl.ds(r0, CH)], out_hbm.at[pl.ds(r0, CH)])


def _dot(a, b):
    return jnp.dot(a, b, preferred_element_type=jnp.float32)


def _mlp_bn(h, Wa, ba, Wb, bb, g, be):
    h = jnp.maximum(_dot(h, Wa) + ba, 0.0)
    h = jnp.maximum(_dot(h, Wb) + bb, 0.0)
    m0 = jnp.mean(h, axis=0, keepdims=True)
    mean = m0 + jnp.mean(h - m0, axis=0, keepdims=True)
    ctr = h - mean
    var = jnp.mean(ctr * ctr, axis=0, keepdims=True)
    return ctr / jnp.sqrt(var + BN_EPS) * g + be


def _layer_body(eps_ref, x_ref, p_ref, Wa_ref, ba_ref, Wb_ref, bb_ref,
                g_ref, be_ref, o_ref):
    h = (1.0 + eps_ref[0, 0]) * x_ref[...] + p_ref[...]
    o_ref[...] = jnp.maximum(
        _mlp_bn(h, Wa_ref[...], ba_ref[...], Wb_ref[...], bb_ref[...],
                g_ref[...], be_ref[...]),
        0.0,
    )


def _final_body(eps_ref, x_ref, p_ref, Wa_ref, ba_ref, Wb_ref, bb_ref,
                g_ref, be_ref, Wl_ref, bl_ref, o_ref):
    h = (1.0 + eps_ref[0, 0]) * x_ref[...] + p_ref[...]
    h = jnp.maximum(
        _mlp_bn(h, Wa_ref[...], ba_ref[...], Wb_ref[...], bb_ref[...],
                g_ref[...], be_ref[...]),
        0.0,
    )
    o_ref[...] = _dot(h, Wl_ref[...]) + bl_ref[...]


def _tc_call(body, n_dense):
    return pl.pallas_call(
        body,
        out_shape=jax.ShapeDtypeStruct((N, D), jnp.float32),
        in_specs=[pl.BlockSpec(memory_space=pltpu.SMEM)]
        + [pl.BlockSpec(memory_space=pltpu.VMEM)] * n_dense,
        out_specs=pl.BlockSpec(memory_space=pltpu.VMEM),
    )


_layer = _tc_call(_layer_body, 8)
_final = _tc_call(_final_body, 10)


def kernel(x, edge_index, eps1, W1a, b1a, W1b, b1b, g1, be1, eps2, W2a, b2a,
           W2b, b2b, g2, be2, eps3, W3a, b3a, W3b, b3b, g3, be3, Wl, bl):
    src = edge_index[0].astype(jnp.int32)
    dst = edge_index[1].astype(jnp.int32)
    pad = SCAN_LEN - E
    ssc = jnp.concatenate([src, jnp.zeros((pad,), jnp.int32)])
    dsc = jnp.concatenate([dst, jnp.full((pad,), SCAN_SENT, jnp.int32)])
    plist, counts = _sc_prepass(ssc, dsc)

    vec = lambda v: jnp.reshape(v, (1, D))
    sca = lambda v: jnp.reshape(v, (1, 1))

    p = _sc_agg(x, plist, counts)
    h = _layer(sca(eps1), x, p, W1a, vec(b1a), W1b, vec(b1b), vec(g1), vec(be1))
    p = _sc_agg(h, plist, counts)
    h = _layer(sca(eps2), h, p, W2a, vec(b2a), W2b, vec(b2b), vec(g2), vec(be2))
    p = _sc_agg(h, plist, counts)
    return _final(sca(eps3), h, p, W3a, vec(b3a), W3b, vec(b3b), vec(g3),
                  vec(be3), Wl, vec(bl))
